# Initial kernel scaffold; baseline (speedup 1.0000x reference)
import jax
import jax.numpy as jnp
from jax.experimental import pallas as pl


def kernel(feat, edge_index):
    raise NotImplementedError("write your pallas kernel here")


# speedup vs baseline: 5.2838x; 5.2838x over previous
import functools

import jax
import jax.numpy as jnp
from jax import lax
from jax.experimental import pallas as pl
from jax.experimental.pallas import tpu as pltpu
from jax.experimental.pallas import tpu_sc as plsc

N_NODES = 10000
N_EDGES = 320000
D_FEAT = 128

NC = 2
NS = 16
NW = NC * NS
EDGES_PER_TILE = N_EDGES // NW
CHUNK = 80
N_CHUNKS = EDGES_PER_TILE // CHUNK
ROWS_PER_TILE = 624
ROWS_LAST = N_NODES - (NS - 1) * ROWS_PER_TILE

_MESH = plsc.VectorSubcoreMesh(core_axis_name="c", subcore_axis_name="s")


@functools.partial(
    pl.kernel,
    out_type=jax.ShapeDtypeStruct((NC, N_NODES, D_FEAT), jnp.float32),
    mesh=_MESH,
    scratch_types=[
        pltpu.VMEM((CHUNK,), jnp.int32),
        pltpu.VMEM((CHUNK,), jnp.int32),
        pltpu.VMEM((CHUNK, D_FEAT), jnp.float32),
        pltpu.VMEM_SHARED((N_NODES, D_FEAT), jnp.float32),
        pltpu.SemaphoreType.DMA,
    ],
)
def _sc_aggregate(feat_hbm, src_hbm, dst_hbm, out_hbm,
                  src_v, dst_v, rows_v, acc_sh, sem):
    c = lax.axis_index("c")
    s = lax.axis_index("s")
    w = c * NS + s

    r0 = pl.multiple_of(s * ROWS_PER_TILE, 8)

    @pl.when(s < NS - 1)
    def _():
        pltpu.sync_copy(feat_hbm.at[pl.ds(r0, ROWS_PER_TILE)],
                        acc_sh.at[pl.ds(r0, ROWS_PER_TILE)])

    @pl.when(s == NS - 1)
    def _():
        pltpu.sync_copy(feat_hbm.at[pl.ds(r0, ROWS_LAST)],
                        acc_sh.at[pl.ds(r0, ROWS_LAST)])

    plsc.subcore_barrier()

    base = w * EDGES_PER_TILE

    def body(i, carry):
        off = pl.multiple_of(base + i * CHUNK, 8)
        pltpu.sync_copy(src_hbm.at[pl.ds(off, CHUNK)], src_v)
        pltpu.sync_copy(dst_hbm.at[pl.ds(off, CHUNK)], dst_v)
        pltpu.async_copy(feat_hbm.at[src_v], rows_v, sem).wait()
        pltpu.sync_copy(rows_v, acc_sh.at[dst_v], add=True)
        return carry

    lax.fori_loop(0, N_CHUNKS, body, 0)
    plsc.subcore_barrier()

    @pl.when(s < NS - 1)
    def _():
        pltpu.sync_copy(acc_sh.at[pl.ds(r0, ROWS_PER_TILE)],
                        out_hbm.at[c, pl.ds(r0, ROWS_PER_TILE)])

    @pl.when(s == NS - 1)
    def _():
        pltpu.sync_copy(acc_sh.at[pl.ds(r0, ROWS_LAST)],
                        out_hbm.at[c, pl.ds(r0, ROWS_LAST)])


_BLK = 400


def _combine_body(f_ref, p0_ref, p1_ref, o_ref):
    o_ref[...] = p0_ref[...] + p1_ref[...] - f_ref[...]


def _combine(feat, p0, p1):
    return pl.pallas_call(
        _combine_body,
        out_shape=jax.ShapeDtypeStruct((N_NODES, D_FEAT), jnp.float32),
        grid=(N_NODES // _BLK,),
        in_specs=[pl.BlockSpec((_BLK, D_FEAT), lambda i: (i, 0))] * 3,
        out_specs=pl.BlockSpec((_BLK, D_FEAT), lambda i: (i, 0)),
    )(feat, p0, p1)


def kernel(feat, edge_index):
    ei = edge_index.astype(jnp.int32)
    src = ei[0]
    dst = ei[1]
    partials = _sc_aggregate(feat, src, dst)
    return _combine(feat, partials[0], partials[1])


# speedup vs baseline: 7.9507x; 1.5047x over previous
import functools

import jax
import jax.numpy as jnp
from jax import lax
from jax.experimental import pallas as pl
from jax.experimental.pallas import tpu as pltpu
from jax.experimental.pallas import tpu_sc as plsc

N_NODES = 10000
N_EDGES = 320000
D_FEAT = 128

NC = 2
NS = 16
NW = NC * NS
EDGES_PER_TILE = N_EDGES // NW
CHUNK = 40
NBUF = 5
N_CHUNKS = EDGES_PER_TILE // CHUNK
N_ROUNDS = N_CHUNKS // NBUF
GROUP = NBUF * CHUNK
ROWS_PER_TILE = 624
ROWS_LAST = N_NODES - (NS - 1) * ROWS_PER_TILE

_MESH = plsc.VectorSubcoreMesh(core_axis_name="c", subcore_axis_name="s")


@functools.partial(
    pl.kernel,
    out_type=jax.ShapeDtypeStruct((NC, N_NODES, D_FEAT), jnp.float32),
    mesh=_MESH,
    scratch_types=[
        pltpu.VMEM((GROUP, D_FEAT), jnp.float32),
        pltpu.VMEM((NBUF, 2, CHUNK), jnp.int32),
        pltpu.VMEM((NBUF, 2, CHUNK), jnp.int32),
        pltpu.VMEM_SHARED((N_NODES, D_FEAT), jnp.float32),
        pltpu.SemaphoreType.DMA,
        pltpu.SemaphoreType.DMA,
        pltpu.SemaphoreType.DMA,
        pltpu.SemaphoreType.DMA,
    ],
)
def _sc_aggregate(feat_hbm, idx_hbm, out_hbm,
                  rows_v, ib0, ib1, acc_sh, gsem, ssem, isem0, isem1):
    c = lax.axis_index("c")
    s = lax.axis_index("s")
    w = c * NS + s
    ibufs = (ib0, ib1)
    isems = (isem0, isem1)

    r0 = pl.multiple_of(s * ROWS_PER_TILE, 8)

    @pl.when(s < NS - 1)
    def _():
        pltpu.sync_copy(feat_hbm.at[pl.ds(r0, ROWS_PER_TILE)],
                        acc_sh.at[pl.ds(r0, ROWS_PER_TILE)])

    @pl.when(s == NS - 1)
    def _():
        pltpu.sync_copy(feat_hbm.at[pl.ds(r0, ROWS_LAST)],
                        acc_sh.at[pl.ds(r0, ROWS_LAST)])

    plsc.subcore_barrier()

    def load_idx(r, p):
        pltpu.async_copy(idx_hbm.at[w, r], ibufs[p], isems[p])

    def wait_idx(p):
        pltpu.make_async_copy(idx_hbm.at[w, 0], ibufs[p], isems[p]).wait()

    def fire_gathers(p):
        for b in range(NBUF):
            pltpu.async_copy(
                feat_hbm.at[ibufs[p].at[b, 0]],
                rows_v.at[pl.ds(b * CHUNK, CHUNK)], gsem)

    def drain(sem):
        pltpu.make_async_copy(feat_hbm.at[pl.ds(0, GROUP)], rows_v, sem).wait()

    def fire_scatters(p):
        for b in range(NBUF):
            pltpu.async_copy(
                rows_v.at[pl.ds(b * CHUNK, CHUNK)],
                acc_sh.at[ibufs[p].at[b, 1]], ssem, add=True)

    load_idx(0, 0)
    load_idx(1, 1)
    wait_idx(0)
    fire_gathers(0)

    def super_body(sr, carry):
        for half in range(2):
            r = 2 * sr + half
            p = half
            drain(gsem)
            fire_scatters(p)
            drain(ssem)
            load_idx(r + 2, p)
            wait_idx(1 - p)
            fire_gathers(1 - p)
        return carry

    lax.fori_loop(0, (N_ROUNDS - 2) // 2, super_body, 0)

    drain(gsem)
    fire_scatters(0)
    drain(ssem)
    wait_idx(1)
    fire_gathers(1)
    drain(gsem)
    fire_scatters(1)
    drain(ssem)

    plsc.subcore_barrier()

    @pl.when(s < NS - 1)
    def _():
        pltpu.sync_copy(acc_sh.at[pl.ds(r0, ROWS_PER_TILE)],
                        out_hbm.at[c, pl.ds(r0, ROWS_PER_TILE)])

    @pl.when(s == NS - 1)
    def _():
        pltpu.sync_copy(acc_sh.at[pl.ds(r0, ROWS_LAST)],
                        out_hbm.at[c, pl.ds(r0, ROWS_LAST)])


_BLK = 400


def _combine_body(f_ref, p0_ref, p1_ref, o_ref):
    o_ref[...] = p0_ref[...] + p1_ref[...] - f_ref[...]


def _combine(feat, p0, p1):
    return pl.pallas_call(
        _combine_body,
        out_shape=jax.ShapeDtypeStruct((N_NODES, D_FEAT), jnp.float32),
        grid=(N_NODES // _BLK,),
        in_specs=[pl.BlockSpec((_BLK, D_FEAT), lambda i: (i, 0))] * 3,
        out_specs=pl.BlockSpec((_BLK, D_FEAT), lambda i: (i, 0)),
    )(feat, p0, p1)


def kernel(feat, edge_index):
    ei = edge_index.astype(jnp.int32)
    src = ei[0].reshape(NW, N_ROUNDS, NBUF, CHUNK)
    dst = ei[1].reshape(NW, N_ROUNDS, NBUF, CHUNK)
    packed = jnp.stack([src, dst], axis=3)
    partials = _sc_aggregate(feat, packed)
    return _combine(feat, partials[0], partials[1])
